# Initial kernel scaffold; baseline (speedup 1.0000x reference)
import jax
import jax.numpy as jnp
from jax.experimental import pallas as pl


def kernel(x, W):
    raise NotImplementedError("write your pallas kernel here")


# speedup vs baseline: 5.1483x; 5.1483x over previous
import functools

import jax
import jax.numpy as jnp
from jax import lax
from jax.experimental import pallas as pl
from jax.experimental.pallas import tpu as pltpu, tpu_sc as plsc

BATCH = 4096
HIST = 200
D = 128
B = BATCH * HIST
ROWS = B // 128
NC = 2
NS = 16
NW = NC * NS
ROWS_PER_W = ROWS // NW
K = 4
NCHUNKS = ROWS_PER_W // K


def _make_emb():
    mesh = plsc.VectorSubcoreMesh(core_axis_name="c", subcore_axis_name="s")

    @functools.partial(
        pl.kernel,
        mesh=mesh,
        out_type=jax.ShapeDtypeStruct((ROWS, 128, D), jnp.float32),
        scratch_types=[
            pltpu.VMEM((K, 128), jnp.int32),
            pltpu.VMEM((K, 128, D), jnp.float32),
            pltpu.SemaphoreType.DMA,
        ],
    )
    def emb(x_hbm, w_hbm, out_hbm, idx_v, rows_v, sem):
        wid = lax.axis_index("s") * NC + lax.axis_index("c")
        row0 = wid * ROWS_PER_W

        def chunk(i, carry):
            r = row0 + i * K
            pltpu.sync_copy(x_hbm.at[pl.ds(r, K)], idx_v)
            copies = [
                pltpu.async_copy(w_hbm.at[idx_v.at[j]], rows_v.at[j], sem)
                for j in range(K)
            ]
            for cp in copies:
                cp.wait()
            pltpu.sync_copy(rows_v, out_hbm.at[pl.ds(r, K)])
            return carry

        lax.fori_loop(0, NCHUNKS, chunk, 0)

    return emb


_emb = _make_emb()


def kernel(x, W):
    x2 = x.reshape(ROWS, 128)
    out = _emb(x2, W)
    return out.reshape(BATCH, HIST, D)


# speedup vs baseline: 5.2169x; 1.0133x over previous
import functools

import jax
import jax.numpy as jnp
from jax import lax
from jax.experimental import pallas as pl
from jax.experimental.pallas import tpu as pltpu, tpu_sc as plsc

BATCH = 4096
HIST = 200
D = 128
B = BATCH * HIST
ROWS = B // 128
NC = 2
NS = 16
NW = NC * NS
ROWS_PER_W = ROWS // NW
K = 2
NCH = ROWS_PER_W // K


def _make_emb():
    mesh = plsc.VectorSubcoreMesh(core_axis_name="c", subcore_axis_name="s")

    @functools.partial(
        pl.kernel,
        mesh=mesh,
        out_type=jax.ShapeDtypeStruct((ROWS, 128, D), jnp.float32),
        scratch_types=[
            pltpu.VMEM((ROWS_PER_W, 128), jnp.int32),
            pltpu.VMEM((2, K, 128, D), jnp.float32),
            pltpu.SemaphoreType.DMA,
            pltpu.SemaphoreType.DMA,
            pltpu.SemaphoreType.DMA,
            pltpu.SemaphoreType.DMA,
        ],
    )
    def emb(x_hbm, w_hbm, out_hbm, idx_v, rows_v, sg0, sg1, ss0, ss1):
        wid = lax.axis_index("s") * NC + lax.axis_index("c")
        row0 = wid * ROWS_PER_W
        sems_g = (sg0, sg1)
        sems_s = (ss0, ss1)

        def fire_gather(c, b):
            for j in range(K):
                pltpu.async_copy(
                    w_hbm.at[idx_v.at[c * K + j]], rows_v.at[b, j], sems_g[b])

        def wait_gather(b):
            for j in range(K):
                pltpu.make_async_copy(
                    w_hbm.at[idx_v.at[0]], rows_v.at[b, j], sems_g[b]).wait()

        def fire_scatter(c, b):
            pltpu.async_copy(
                rows_v.at[b], out_hbm.at[pl.ds(row0 + c * K, K)], sems_s[b])

        def wait_scatter(b):
            pltpu.make_async_copy(
                rows_v.at[b], out_hbm.at[pl.ds(row0, K)], sems_s[b]).wait()

        pltpu.sync_copy(x_hbm.at[pl.ds(row0, ROWS_PER_W)], idx_v)

        fire_gather(0, 0)
        wait_gather(0)
        fire_gather(1, 1)
        fire_scatter(0, 0)

        def group(g, carry):
            c = 2 * g + 1
            wait_gather(1)
            wait_scatter(0)
            fire_gather(c + 1, 0)
            fire_scatter(c, 1)
            wait_gather(0)
            wait_scatter(1)
            fire_gather(c + 2, 1)
            fire_scatter(c + 1, 0)
            return carry

        lax.fori_loop(0, (NCH - 2) // 2, group, 0)

        wait_gather(1)
        wait_scatter(0)
        fire_scatter(NCH - 1, 1)
        wait_scatter(1)

    return emb


_emb = _make_emb()


def kernel(x, W):
    x2 = x.reshape(ROWS, 128)
    out = _emb(x2, W)
    return out.reshape(BATCH, HIST, D)


# speedup vs baseline: 15.4704x; 2.9654x over previous
import functools

import jax
import jax.numpy as jnp
from jax import lax
from jax.experimental import pallas as pl
from jax.experimental.pallas import tpu as pltpu, tpu_sc as plsc

BATCH = 4096
HIST = 200
D = 128
B = BATCH * HIST
ROWS = B // 128
NC = 2
NS = 16
NW = NC * NS
ROWS_PER_W = ROWS // NW
K = 2
NCH = ROWS_PER_W // K


def _make_emb():
    mesh = plsc.VectorSubcoreMesh(core_axis_name="c", subcore_axis_name="s")

    @functools.partial(
        pl.kernel,
        mesh=mesh,
        out_type=jax.ShapeDtypeStruct((ROWS, 128, D), jnp.float32),
        scratch_types=[
            pltpu.VMEM((ROWS_PER_W, 128), jnp.int32),
            pltpu.VMEM((2, K, 128, D), jnp.float32),
            pltpu.VMEM_SHARED((1000, D), jnp.float32),
            pltpu.SemaphoreType.DMA,
            pltpu.SemaphoreType.DMA,
            pltpu.SemaphoreType.DMA,
            pltpu.SemaphoreType.DMA,
        ],
    )
    def emb(x_hbm, w_hbm, out_hbm, idx_v, rows_v, w_sp, sg0, sg1, ss0, ss1):
        wid = lax.axis_index("s") * NC + lax.axis_index("c")
        row0 = wid * ROWS_PER_W
        sems_g = (sg0, sg1)
        sems_s = (ss0, ss1)

        @pl.when(lax.axis_index("s") == 0)
        def _():
            pltpu.sync_copy(w_hbm, w_sp)

        plsc.subcore_barrier()

        def fire_gather(c, b):
            for j in range(K):
                pltpu.async_copy(
                    w_sp.at[idx_v.at[c * K + j]], rows_v.at[b, j], sems_g[b])

        def wait_gather(b):
            for j in range(K):
                pltpu.make_async_copy(
                    w_sp.at[idx_v.at[0]], rows_v.at[b, j], sems_g[b]).wait()

        def fire_scatter(c, b):
            pltpu.async_copy(
                rows_v.at[b], out_hbm.at[pl.ds(row0 + c * K, K)], sems_s[b])

        def wait_scatter(b):
            pltpu.make_async_copy(
                rows_v.at[b], out_hbm.at[pl.ds(row0, K)], sems_s[b]).wait()

        pltpu.sync_copy(x_hbm.at[pl.ds(row0, ROWS_PER_W)], idx_v)

        fire_gather(0, 0)
        wait_gather(0)
        fire_gather(1, 1)
        fire_scatter(0, 0)

        def group(g, carry):
            c = 2 * g + 1
            wait_gather(1)
            wait_scatter(0)
            fire_gather(c + 1, 0)
            fire_scatter(c, 1)
            wait_gather(0)
            wait_scatter(1)
            fire_gather(c + 2, 1)
            fire_scatter(c + 1, 0)
            return carry

        lax.fori_loop(0, (NCH - 2) // 2, group, 0)

        wait_gather(1)
        wait_scatter(0)
        fire_scatter(NCH - 1, 1)
        wait_scatter(1)

    return emb


_emb = _make_emb()


def kernel(x, W):
    x2 = x.reshape(ROWS, 128)
    out = _emb(x2, W)
    return out.reshape(BATCH, HIST, D)


# speedup vs baseline: 15.8915x; 1.0272x over previous
import functools

import jax
import jax.numpy as jnp
from jax import lax
from jax.experimental import pallas as pl
from jax.experimental.pallas import tpu as pltpu, tpu_sc as plsc

BATCH = 4096
HIST = 200
D = 128
B = BATCH * HIST
ROWS = B // 128
NC = 2
NS = 16
NW = NC * NS
ROWS_PER_W = ROWS // NW
K = 1
NCH = ROWS_PER_W // K
NBUF = 4


def _make_emb():
    mesh = plsc.VectorSubcoreMesh(core_axis_name="c", subcore_axis_name="s")

    @functools.partial(
        pl.kernel,
        mesh=mesh,
        out_type=jax.ShapeDtypeStruct((ROWS, 128, D), jnp.float32),
        scratch_types=[
            pltpu.VMEM((ROWS_PER_W, 128), jnp.int32),
            pltpu.VMEM((NBUF, K, 128, D), jnp.float32),
            pltpu.VMEM_SHARED((1000, D), jnp.float32),
            pltpu.SemaphoreType.DMA,
            pltpu.SemaphoreType.DMA,
            pltpu.SemaphoreType.DMA,
            pltpu.SemaphoreType.DMA,
            pltpu.SemaphoreType.DMA,
            pltpu.SemaphoreType.DMA,
            pltpu.SemaphoreType.DMA,
            pltpu.SemaphoreType.DMA,
        ],
    )
    def emb(x_hbm, w_hbm, out_hbm, idx_v, rows_v, w_sp,
            sg0, sg1, sg2, sg3, ss0, ss1, ss2, ss3):
        wid = lax.axis_index("s") * NC + lax.axis_index("c")
        row0 = wid * ROWS_PER_W
        sems_g = (sg0, sg1, sg2, sg3)
        sems_s = (ss0, ss1, ss2, ss3)

        @pl.when(lax.axis_index("s") == 0)
        def _():
            pltpu.sync_copy(w_hbm, w_sp)

        plsc.subcore_barrier()

        def fire_gather(c, b):
            for j in range(K):
                pltpu.async_copy(
                    w_sp.at[idx_v.at[c * K + j]], rows_v.at[b, j], sems_g[b])

        def wait_gather(b):
            for j in range(K):
                pltpu.make_async_copy(
                    w_sp.at[idx_v.at[0]], rows_v.at[b, j], sems_g[b]).wait()

        def fire_scatter(c, b):
            pltpu.async_copy(
                rows_v.at[b], out_hbm.at[pl.ds(row0 + c * K, K)], sems_s[b])

        def wait_scatter(b):
            pltpu.make_async_copy(
                rows_v.at[b], out_hbm.at[pl.ds(row0, K)], sems_s[b]).wait()

        pltpu.sync_copy(x_hbm.at[pl.ds(row0, ROWS_PER_W)], idx_v)

        def step(c, b):
            wait_gather(b)
            fire_scatter(c, b)
            wait_scatter((b + 2) % NBUF)
            fire_gather(c + 2, (b + 2) % NBUF)

        fire_gather(0, 0)
        fire_gather(1, 1)
        wait_gather(0)
        fire_scatter(0, 0)
        fire_gather(2, 2)
        wait_gather(1)
        fire_scatter(1, 1)
        fire_gather(3, 3)

        def group(g, carry):
            c = NBUF * g + 2
            step(c, 2)
            step(c + 1, 3)
            step(c + 2, 0)
            step(c + 3, 1)
            return carry

        lax.fori_loop(0, (NCH - 4) // NBUF, group, 0)

        wait_gather(2)
        fire_scatter(NCH - 2, 2)
        wait_gather(3)
        fire_scatter(NCH - 1, 3)
        wait_scatter(0)
        wait_scatter(1)
        wait_scatter(2)
        wait_scatter(3)

    return emb


_emb = _make_emb()


def kernel(x, W):
    x2 = x.reshape(ROWS, 128)
    out = _emb(x2, W)
    return out.reshape(BATCH, HIST, D)


# speedup vs baseline: 15.9141x; 1.0014x over previous
import functools

import jax
import jax.numpy as jnp
from jax import lax
from jax.experimental import pallas as pl
from jax.experimental.pallas import tpu as pltpu, tpu_sc as plsc

BATCH = 4096
HIST = 200
D = 128
B = BATCH * HIST
ROWS = B // 128
NC = 2
NS = 16
NW = NC * NS
ROWS_PER_W = ROWS // NW
NCH = ROWS_PER_W
NBUF = 5


def _make_emb():
    mesh = plsc.VectorSubcoreMesh(core_axis_name="c", subcore_axis_name="s")

    @functools.partial(
        pl.kernel,
        mesh=mesh,
        out_type=jax.ShapeDtypeStruct((ROWS, 128, D), jnp.float32),
        scratch_types=[
            pltpu.VMEM((ROWS_PER_W, 128), jnp.int32),
            pltpu.VMEM((NBUF, 128, D), jnp.float32),
            pltpu.VMEM_SHARED((1000, D), jnp.float32),
            pltpu.SemaphoreType.DMA,
            pltpu.SemaphoreType.DMA,
            pltpu.SemaphoreType.DMA,
            pltpu.SemaphoreType.DMA,
            pltpu.SemaphoreType.DMA,
            pltpu.SemaphoreType.DMA,
            pltpu.SemaphoreType.DMA,
            pltpu.SemaphoreType.DMA,
            pltpu.SemaphoreType.DMA,
            pltpu.SemaphoreType.DMA,
            pltpu.SemaphoreType.DMA,
        ],
    )
    def emb(x_hbm, w_hbm, out_hbm, idx_v, rows_v, w_sp,
            sg0, sg1, sg2, sg3, sg4, ss0, ss1, ss2, ss3, ss4, si):
        wid = lax.axis_index("s") * NC + lax.axis_index("c")
        row0 = wid * ROWS_PER_W
        sems_g = (sg0, sg1, sg2, sg3, sg4)
        sems_s = (ss0, ss1, ss2, ss3, ss4)

        idx_cp = pltpu.async_copy(x_hbm.at[pl.ds(row0, ROWS_PER_W)], idx_v, si)

        @pl.when(lax.axis_index("s") == 0)
        def _():
            pltpu.sync_copy(w_hbm, w_sp)

        idx_cp.wait()
        plsc.subcore_barrier()

        def fire_gather(c, b):
            pltpu.async_copy(w_sp.at[idx_v.at[c]], rows_v.at[b], sems_g[b])

        def wait_gather(b):
            pltpu.make_async_copy(
                w_sp.at[idx_v.at[0]], rows_v.at[b], sems_g[b]).wait()

        def fire_scatter(c, b):
            pltpu.async_copy(rows_v.at[b], out_hbm.at[row0 + c], sems_s[b])

        def wait_scatter(b):
            pltpu.make_async_copy(
                rows_v.at[b], out_hbm.at[row0], sems_s[b]).wait()

        def step(c, b):
            wait_gather(b)
            fire_scatter(c, b)
            wait_scatter((b + 2) % NBUF)
            fire_gather(c + 2, (b + 2) % NBUF)

        fire_gather(0, 0)
        fire_gather(1, 1)
        for c in range(NBUF - 2):
            wait_gather(c)
            fire_scatter(c, c)
            fire_gather(c + 2, c + 2)

        def group(g, carry):
            c0 = NBUF * g + (NBUF - 2)
            for i in range(NBUF):
                step(c0 + i, (NBUF - 2 + i) % NBUF)
            return carry

        lax.fori_loop(0, (NCH - NBUF) // NBUF, group, 0)

        wait_gather((NCH - 2) % NBUF)
        fire_scatter(NCH - 2, (NCH - 2) % NBUF)
        wait_gather((NCH - 1) % NBUF)
        fire_scatter(NCH - 1, (NCH - 1) % NBUF)
        for b in range(NBUF):
            wait_scatter(b)

    return emb


_emb = _make_emb()


def kernel(x, W):
    x2 = x.reshape(ROWS, 128)
    out = _emb(x2, W)
    return out.reshape(BATCH, HIST, D)


# speedup vs baseline: 16.0630x; 1.0094x over previous
import functools

import jax
import jax.numpy as jnp
from jax import lax
from jax.experimental import pallas as pl
from jax.experimental.pallas import tpu as pltpu, tpu_sc as plsc

BATCH = 4096
HIST = 200
D = 128
B = BATCH * HIST
ROWS = B // 128
NC = 2
NS = 16
NW = NC * NS
ROWS_PER_W = ROWS // NW
NCH = ROWS_PER_W
NBUF = 5


def _make_emb():
    mesh = plsc.VectorSubcoreMesh(core_axis_name="c", subcore_axis_name="s")

    @functools.partial(
        pl.kernel,
        mesh=mesh,
        out_type=jax.ShapeDtypeStruct((ROWS, 128, D), jnp.float32),
        scratch_types=[
            pltpu.VMEM((ROWS_PER_W, 128), jnp.int32),
            pltpu.VMEM((NBUF, 128, D), jnp.float32),
            pltpu.VMEM_SHARED((1000, D), jnp.float32),
            pltpu.SemaphoreType.DMA,
            pltpu.SemaphoreType.DMA,
            pltpu.SemaphoreType.DMA,
            pltpu.SemaphoreType.DMA,
            pltpu.SemaphoreType.DMA,
            pltpu.SemaphoreType.DMA,
            pltpu.SemaphoreType.DMA,
            pltpu.SemaphoreType.DMA,
            pltpu.SemaphoreType.DMA,
            pltpu.SemaphoreType.DMA,
            pltpu.SemaphoreType.DMA,
        ],
    )
    def emb(x_hbm, w_hbm, out_hbm, idx_v, rows_v, w_sp,
            sg0, sg1, sg2, sg3, sg4, ss0, ss1, ss2, ss3, ss4, si):
        wid = lax.axis_index("s") * NC + lax.axis_index("c")
        row0 = wid * ROWS_PER_W
        sems_g = (sg0, sg1, sg2, sg3, sg4)
        sems_s = (ss0, ss1, ss2, ss3, ss4)

        idx_cp = pltpu.async_copy(x_hbm.at[pl.ds(row0, ROWS_PER_W)], idx_v, si)

        @pl.when(lax.axis_index("s") == 0)
        def _():
            pltpu.sync_copy(w_hbm, w_sp)

        idx_cp.wait()
        plsc.subcore_barrier()

        def fire_gather(c, b):
            pltpu.async_copy(w_sp.at[idx_v.at[c]], rows_v.at[b], sems_g[b])

        def wait_gather(b):
            pltpu.make_async_copy(
                w_sp.at[idx_v.at[0]], rows_v.at[b], sems_g[b]).wait()

        def fire_scatter(c, b):
            pltpu.async_copy(rows_v.at[b], out_hbm.at[row0 + c], sems_s[b])

        def wait_scatter(b):
            pltpu.make_async_copy(
                rows_v.at[b], out_hbm.at[row0], sems_s[b]).wait()

        def step(c, b):
            wait_gather(b)
            fire_scatter(c, b)
            wait_scatter((b + 3) % NBUF)
            fire_gather(c + 3, (b + 3) % NBUF)

        fire_gather(0, 0)
        fire_gather(1, 1)
        fire_gather(2, 2)
        for c in range(NBUF - 3):
            wait_gather(c)
            fire_scatter(c, c)
            fire_gather(c + 3, c + 3)

        def group(g, carry):
            c0 = NBUF * g + (NBUF - 3)
            for i in range(NBUF):
                step(c0 + i, (NBUF - 3 + i) % NBUF)
            return carry

        lax.fori_loop(0, (NCH - NBUF) // NBUF, group, 0)

        for c in range(NCH - 3, NCH):
            wait_gather(c % NBUF)
            fire_scatter(c, c % NBUF)
        for b in range(NBUF):
            wait_scatter(b)

    return emb


_emb = _make_emb()


def kernel(x, W):
    x2 = x.reshape(ROWS, 128)
    out = _emb(x2, W)
    return out.reshape(BATCH, HIST, D)


# speedup vs baseline: 16.0715x; 1.0005x over previous
import functools

import jax
import jax.numpy as jnp
from jax import lax
from jax.experimental import pallas as pl
from jax.experimental.pallas import tpu as pltpu, tpu_sc as plsc

BATCH = 4096
HIST = 200
D = 128
B = BATCH * HIST
ROWS = B // 128
NC = 2
NS = 16
NW = NC * NS
ROWS_PER_W = ROWS // NW
NCH = ROWS_PER_W
NBUF = 5


def _make_emb():
    mesh = plsc.VectorSubcoreMesh(core_axis_name="c", subcore_axis_name="s")

    @functools.partial(
        pl.kernel,
        mesh=mesh,
        out_type=jax.ShapeDtypeStruct((ROWS, 128, D), jnp.float32),
        scratch_types=[
            pltpu.VMEM((ROWS_PER_W, 128), jnp.int32),
            pltpu.VMEM((NBUF, 128, D), jnp.float32),
            pltpu.VMEM_SHARED((1000, D), jnp.float32),
            pltpu.SemaphoreType.DMA,
            pltpu.SemaphoreType.DMA,
            pltpu.SemaphoreType.DMA,
            pltpu.SemaphoreType.DMA,
            pltpu.SemaphoreType.DMA,
            pltpu.SemaphoreType.DMA,
            pltpu.SemaphoreType.DMA,
            pltpu.SemaphoreType.DMA,
            pltpu.SemaphoreType.DMA,
            pltpu.SemaphoreType.DMA,
            pltpu.SemaphoreType.DMA,
        ],
    )
    def emb(x_hbm, w_hbm, out_hbm, idx_v, rows_v, w_sp,
            sg0, sg1, sg2, sg3, sg4, ss0, ss1, ss2, ss3, ss4, si):
        wid = lax.axis_index("s") * NC + lax.axis_index("c")
        row0 = wid * ROWS_PER_W
        sems_g = (sg0, sg1, sg2, sg3, sg4)
        sems_s = (ss0, ss1, ss2, ss3, ss4)

        idx_cp = pltpu.async_copy(x_hbm.at[pl.ds(row0, ROWS_PER_W)], idx_v, si)

        @pl.when(lax.axis_index("s") == 0)
        def _():
            pltpu.sync_copy(w_hbm, w_sp)

        idx_cp.wait()
        plsc.subcore_barrier()

        def fire_gather(c, b):
            pltpu.async_copy(w_sp.at[idx_v.at[c]], rows_v.at[b], sems_g[b])

        def wait_gather(b):
            pltpu.make_async_copy(
                w_sp.at[idx_v.at[0]], rows_v.at[b], sems_g[b]).wait()

        def fire_scatter(c, b):
            pltpu.async_copy(rows_v.at[b], out_hbm.at[row0 + c], sems_s[b])

        def wait_scatter(b):
            pltpu.make_async_copy(
                rows_v.at[b], out_hbm.at[row0], sems_s[b]).wait()

        def step(c, b):
            wait_scatter((b + 3) % NBUF)
            fire_gather(c + 3, (b + 3) % NBUF)
            wait_gather(b)
            fire_scatter(c, b)

        fire_gather(0, 0)
        fire_gather(1, 1)
        fire_gather(2, 2)
        for c in range(NBUF - 3):
            wait_gather(c)
            fire_scatter(c, c)
            fire_gather(c + 3, c + 3)

        def group(g, carry):
            c0 = NBUF * g + (NBUF - 3)
            for i in range(NBUF):
                step(c0 + i, (NBUF - 3 + i) % NBUF)
            return carry

        lax.fori_loop(0, (NCH - NBUF) // NBUF, group, 0)

        for c in range(NCH - 3, NCH):
            wait_gather(c % NBUF)
            fire_scatter(c, c % NBUF)
        for b in range(NBUF):
            wait_scatter(b)

    return emb


_emb = _make_emb()


def kernel(x, W):
    x2 = x.reshape(ROWS, 128)
    out = _emb(x2, W)
    return out.reshape(BATCH, HIST, D)


# speedup vs baseline: 16.0720x; 1.0000x over previous
import functools

import jax
import jax.numpy as jnp
from jax import lax
from jax.experimental import pallas as pl
from jax.experimental.pallas import tpu as pltpu, tpu_sc as plsc

BATCH = 4096
HIST = 200
D = 128
B = BATCH * HIST
ROWS = B // 128
NC = 2
NS = 16
NW = NC * NS
ROWS_PER_W = ROWS // NW
NCH = ROWS_PER_W
NBUF = 5


def _make_emb():
    mesh = plsc.VectorSubcoreMesh(core_axis_name="c", subcore_axis_name="s")

    @functools.partial(
        pl.kernel,
        mesh=mesh,
        out_type=jax.ShapeDtypeStruct((ROWS, 128, D), jnp.float32),
        scratch_types=[
            pltpu.VMEM((ROWS_PER_W, 128), jnp.int32),
            pltpu.VMEM((NBUF, 128, D), jnp.float32),
            pltpu.VMEM_SHARED((1000, D), jnp.float32),
            pltpu.SemaphoreType.DMA,
            pltpu.SemaphoreType.DMA,
            pltpu.SemaphoreType.DMA,
            pltpu.SemaphoreType.DMA,
            pltpu.SemaphoreType.DMA,
            pltpu.SemaphoreType.DMA,
            pltpu.SemaphoreType.DMA,
            pltpu.SemaphoreType.DMA,
            pltpu.SemaphoreType.DMA,
            pltpu.SemaphoreType.DMA,
            pltpu.SemaphoreType.DMA,
        ],
    )
    def emb(x_hbm, w_hbm, out_hbm, idx_v, rows_v, w_sp,
            sg0, sg1, sg2, sg3, sg4, ss0, ss1, ss2, ss3, ss4, si):
        wid = lax.axis_index("s") * NC + lax.axis_index("c")
        row0 = wid * ROWS_PER_W
        sems_g = (sg0, sg1, sg2, sg3, sg4)
        sems_s = (ss0, ss1, ss2, ss3, ss4)

        idx_cp = pltpu.async_copy(x_hbm.at[pl.ds(row0, ROWS_PER_W)], idx_v, si)

        @pl.when(lax.axis_index("s") == 0)
        def _():
            pltpu.sync_copy(w_hbm, w_sp)

        idx_cp.wait()
        plsc.subcore_barrier()

        def fire_gather(c, b):
            pltpu.async_copy(w_sp.at[idx_v.at[c]], rows_v.at[b], sems_g[b])

        def wait_gather(b):
            pltpu.make_async_copy(
                w_sp.at[idx_v.at[0]], rows_v.at[b], sems_g[b]).wait()

        def fire_scatter(c, b):
            pltpu.async_copy(rows_v.at[b], out_hbm.at[row0 + c], sems_s[b])

        def wait_scatter(b):
            pltpu.make_async_copy(
                rows_v.at[b], out_hbm.at[row0], sems_s[b]).wait()

        def step(c, b):
            wait_gather(b)
            fire_scatter(c, b)
            wait_scatter((b + 3) % NBUF)
            fire_gather(c + 3, (b + 3) % NBUF)

        fire_gather(0, 0)
        fire_gather(1, 1)
        fire_gather(2, 2)
        for c in range(NBUF - 3):
            wait_gather(c)
            fire_scatter(c, c)
            fire_gather(c + 3, c + 3)

        def group(g, carry):
            c0 = NBUF * g + (NBUF - 3)
            for i in range(NBUF):
                step(c0 + i, (NBUF - 3 + i) % NBUF)
            return carry

        lax.fori_loop(0, (NCH - NBUF) // NBUF, group, 0)

        for c in range(NCH - 3, NCH):
            wait_gather(c % NBUF)
            fire_scatter(c, c % NBUF)
        for b in range(NBUF):
            wait_scatter(b)

    return emb


_emb = _make_emb()


def kernel(x, W):
    x2 = x.reshape(ROWS, 128)
    out = _emb(x2, W)
    return out.reshape(BATCH, HIST, D)
